# baseline (speedup 1.0000x reference)
import functools

import jax
import jax.numpy as jnp
from jax import lax
from jax.experimental import pallas as pl
from jax.experimental.pallas import tpu as pltpu
from jax.experimental.pallas import tpu_sc as plsc

BATCH = 16384
D = 32
NC = 2
NS = 16
NW = NC * NS
BPW = BATCH // NW
CHUNK = 128
NCHUNK = BPW // CHUNK

_sc_mesh = plsc.VectorSubcoreMesh(core_axis_name="c", subcore_axis_name="s")


@functools.partial(
    pl.kernel,
    mesh=_sc_mesh,
    out_type=(
        jax.ShapeDtypeStruct((BATCH, D), jnp.float32),
        jax.ShapeDtypeStruct((BATCH, D), jnp.float32),
    ),
    scratch_types=[
        pltpu.VMEM((NCHUNK, CHUNK), jnp.int32),
        pltpu.VMEM((NCHUNK, CHUNK), jnp.int32),
        pltpu.VMEM((BPW, D), jnp.float32),
        pltpu.VMEM((BPW, D), jnp.float32),
        pltpu.SemaphoreType.DMA,
        pltpu.SemaphoreType.DMA,
    ],
    compiler_params=pltpu.CompilerParams(use_tc_tiling_on_sc=False),
)
def _sc_gather(uid_hbm, iid_hbm, ut_hbm, it_hbm, ue_hbm, ie_hbm,
               uidx_v, iidx_v, urows_v, irows_v, sem_u, sem_i):
    wid = lax.axis_index("s") * NC + lax.axis_index("c")
    base = wid * BPW
    pltpu.sync_copy(uid_hbm.at[wid], uidx_v)
    pltpu.sync_copy(iid_hbm.at[wid], iidx_v)
    copies = []
    for c in range(NCHUNK):
        copies.append(pltpu.async_copy(
            ut_hbm.at[uidx_v.at[c]], urows_v.at[pl.ds(c * CHUNK, CHUNK)], sem_u))
        copies.append(pltpu.async_copy(
            it_hbm.at[iidx_v.at[c]], irows_v.at[pl.ds(c * CHUNK, CHUNK)], sem_i))
    for cp in copies:
        cp.wait()
    pltpu.sync_copy(urows_v, ue_hbm.at[pl.ds(base, BPW)])
    pltpu.sync_copy(irows_v, ie_hbm.at[pl.ds(base, BPW)])


BB = 2048
NPAD = 128


def _mlp_body(ue_ref, ie_ref, w1u_ref, w1i_ref, b1_ref, w2_ref, b2_ref, out_ref):
    h = jnp.dot(ue_ref[...], w1u_ref[...], preferred_element_type=jnp.float32)
    h = h + jnp.dot(ie_ref[...], w1i_ref[...], preferred_element_type=jnp.float32)
    h = jnp.maximum(h + b1_ref[...], 0.0)
    logits = jnp.dot(h, w2_ref[...], preferred_element_type=jnp.float32) + b2_ref[...]
    lane = lax.broadcasted_iota(jnp.int32, logits.shape, 1)
    masked = jnp.where(lane < 5, logits, -jnp.inf)
    m = jnp.max(masked, axis=1, keepdims=True)
    e = jnp.exp(masked - m)
    s = jnp.sum(e, axis=1, keepdims=True)
    out_ref[...] = (e / s)[:, :5]


def _mlp(ue, ie, w1u, w1i, b1, w2p, b2p):
    grid = (BATCH // BB,)
    return pl.pallas_call(
        _mlp_body,
        grid=grid,
        in_specs=[
            pl.BlockSpec((BB, D), lambda i: (i, 0)),
            pl.BlockSpec((BB, D), lambda i: (i, 0)),
            pl.BlockSpec((D, 64), lambda i: (0, 0)),
            pl.BlockSpec((D, 64), lambda i: (0, 0)),
            pl.BlockSpec((1, 64), lambda i: (0, 0)),
            pl.BlockSpec((64, NPAD), lambda i: (0, 0)),
            pl.BlockSpec((1, NPAD), lambda i: (0, 0)),
        ],
        out_specs=pl.BlockSpec((BB, 5), lambda i: (i, 0)),
        out_shape=jax.ShapeDtypeStruct((BATCH, 5), jnp.float32),
    )(ue, ie, w1u, w1i, b1, w2p, b2p)


def kernel(user_ids, item_ids, user_table, item_table, W1, b1, W2, b2):
    uid = jnp.reshape(user_ids.astype(jnp.int32), (NW, NCHUNK, CHUNK))
    iid = jnp.reshape(item_ids.astype(jnp.int32), (NW, NCHUNK, CHUNK))
    ue, ie = _sc_gather(uid, iid, user_table, item_table)
    w1u = jnp.transpose(W1[:, :D])
    w1i = jnp.transpose(W1[:, D:])
    w2p = jnp.pad(jnp.transpose(W2), ((0, 0), (0, NPAD - 5)))
    b2p = jnp.pad(jnp.reshape(b2, (1, 5)), ((0, 0), (0, NPAD - 5)))
    return _mlp(ue, ie, w1u, w1i, jnp.reshape(b1, (1, 64)), w2p, b2p)
